# baseline (speedup 1.0000x reference)
import jax
import jax.numpy as jnp
from jax import lax
from jax.experimental import pallas as pl
from jax.experimental.pallas import tpu as pltpu
from jax.experimental.pallas import tpu_sc as plsc

_D = 128
_B = 4096 * 50
_NC, _NS = 2, 16
_NW = _NC * _NS
_BPW = _B // _NW
_CHUNK = 128
_NCHUNK = _BPW // _CHUNK
_NBUF = 5
_NGROUP = _NCHUNK // _NBUF


def _emb_body(idx_hbm, table_hbm, out_hbm, idx_v, bufs, sems, wsems):
    wid = lax.axis_index("s") * _NC + lax.axis_index("c")
    base = wid * _BPW
    pltpu.sync_copy(idx_hbm.at[wid], idx_v)

    def out_slice(j):
        return out_hbm.at[pl.ds(base + j * _CHUNK, _CHUNK)]

    def group(g, carry):
        @pl.when(g > 0)
        def _():
            for b in range(_NBUF):
                pltpu.make_async_copy(
                    bufs.at[b], out_slice(0), wsems.at[b]).wait()
        for b in range(_NBUF):
            j = g * _NBUF + b
            pltpu.async_copy(
                table_hbm.at[idx_v.at[j]], bufs.at[b], sems.at[b])
        for b in range(_NBUF):
            j = g * _NBUF + b
            pltpu.make_async_copy(
                table_hbm.at[idx_v.at[b]], bufs.at[b], sems.at[b]).wait()
            pltpu.async_copy(bufs.at[b], out_slice(j), wsems.at[b])
        return carry

    lax.fori_loop(0, _NGROUP, group, 0, unroll=False)

    for b in range(_NBUF):
        pltpu.make_async_copy(bufs.at[b], out_slice(0), wsems.at[b]).wait()


def kernel(input, table):
    idx = input.reshape(_NW, _NCHUNK, _CHUNK).astype(jnp.int32)
    mesh = plsc.VectorSubcoreMesh(
        core_axis_name="c", subcore_axis_name="s",
        num_cores=_NC, num_subcores=_NS)
    out = pl.kernel(
        _emb_body,
        out_type=jax.ShapeDtypeStruct((_B, _D), jnp.float32),
        mesh=mesh,
        scratch_types=[
            pltpu.VMEM((_NCHUNK, _CHUNK), jnp.int32),
            pltpu.VMEM((_NBUF, _CHUNK, _D), jnp.float32),
            pltpu.SemaphoreType.DMA((_NBUF,)),
            pltpu.SemaphoreType.DMA((_NBUF,)),
        ],
    )(idx, table)
    return out.reshape(input.shape[0], input.shape[1], _D)


# speedup vs baseline: 1.7653x; 1.7653x over previous
import jax
import jax.numpy as jnp
from jax import lax
from jax.experimental import pallas as pl
from jax.experimental.pallas import tpu as pltpu
from jax.experimental.pallas import tpu_sc as plsc

_D = 128
_L = 50
_BATCH = 4096
_NC, _NS = 2, 16
_NW = _NC * _NS
_RPW = _BATCH // _NW
_RCH = 4
_NCHUNK = _RPW // _RCH
_NBUF = 4
_NGROUP = _NCHUNK // _NBUF


def _emb_body(idx_hbm, table_hbm, out_hbm, idx_v, bufs, gsems, wsems):
    wid = lax.axis_index("s") * _NC + lax.axis_index("c")
    row0 = wid * _RPW
    pltpu.sync_copy(idx_hbm.at[pl.ds(row0, _RPW)], idx_v)

    def group(g, carry):
        @pl.when(g > 0)
        def _():
            for b in range(_NBUF):
                pltpu.make_async_copy(
                    bufs.at[b], out_hbm.at[pl.ds(row0, _RCH)],
                    wsems.at[b]).wait()
        for b in range(_NBUF):
            j = g * _NBUF + b
            for k in range(_RCH):
                pltpu.async_copy(
                    table_hbm.at[idx_v.at[j * _RCH + k]],
                    bufs.at[b].at[k], gsems.at[b])
        for b in range(_NBUF):
            j = g * _NBUF + b
            for k in range(_RCH):
                pltpu.make_async_copy(
                    table_hbm.at[idx_v.at[k]], bufs.at[b].at[k],
                    gsems.at[b]).wait()
            pltpu.async_copy(
                bufs.at[b], out_hbm.at[pl.ds(row0 + j * _RCH, _RCH)],
                wsems.at[b])
        return carry

    lax.fori_loop(0, _NGROUP, group, 0, unroll=False)

    for b in range(_NBUF):
        pltpu.make_async_copy(
            bufs.at[b], out_hbm.at[pl.ds(row0, _RCH)], wsems.at[b]).wait()


def kernel(input, table):
    mesh = plsc.VectorSubcoreMesh(
        core_axis_name="c", subcore_axis_name="s",
        num_cores=_NC, num_subcores=_NS)
    return pl.kernel(
        _emb_body,
        out_type=jax.ShapeDtypeStruct((_BATCH, _L, _D), jnp.float32),
        mesh=mesh,
        scratch_types=[
            pltpu.VMEM((_RPW, _L), jnp.int32),
            pltpu.VMEM((_NBUF, _RCH, _L, _D), jnp.float32),
            pltpu.SemaphoreType.DMA((_NBUF,)),
            pltpu.SemaphoreType.DMA((_NBUF,)),
        ],
    )(input.astype(jnp.int32), table)
